# baseline (speedup 1.0000x reference)
import functools

import jax
import jax.numpy as jnp
from jax import lax
from jax.experimental import pallas as pl
from jax.experimental.pallas import tpu as pltpu
from jax.experimental.pallas import tpu_sc as plsc

B = 4096
VOCAB = 1000000
S = 200
D = 64
C = 2
L = 16
SP = 224
NC = 2
NS = 16
NW = NC * NS
RW = B // NW
NCH = SP // L
UN = 8

_mesh = plsc.VectorSubcoreMesh(
    core_axis_name="c", subcore_axis_name="s", num_cores=NC, num_subcores=NS
)


@functools.partial(
    pl.kernel,
    out_type=jax.ShapeDtypeStruct((B, D), jnp.float32),
    mesh=_mesh,
    scratch_types=[
        pltpu.VMEM((RW, SP), jnp.int32),
        pltpu.VMEM((RW, SP), jnp.int32),
        pltpu.VMEM((4 * SP, D // 2), jnp.float32),
        pltpu.VMEM((RW, D), jnp.float32),
        pltpu.SemaphoreType.DMA,
        pltpu.SemaphoreType.DMA,
        pltpu.SemaphoreType.DMA,
        pltpu.SemaphoreType.DMA,
    ],
    compiler_params=pltpu.CompilerParams(
        use_tc_tiling_on_sc=False, needs_layout_passes=False
    ),
)
def _sc_pool(ids_hbm, mask_hbm, emb_hbm, sum_hbm, ids_v, mask_v, buf_v, out_v,
             sem0, sem1, sem2, sem3):
    wid = lax.axis_index("s") * NC + lax.axis_index("c")
    base = wid * RW
    sems = (sem0, sem1, sem2, sem3)

    pltpu.sync_copy(ids_hbm.at[pl.ds(base, RW)], ids_v)
    pltpu.sync_copy(mask_hbm.at[pl.ds(base, RW)], mask_v)

    def compact(r):
        rv = jnp.broadcast_to(r.astype(jnp.int32), (L,))
        off = jnp.int32(0)
        for k in range(NCH):
            sl = pl.ds(k * L, L)
            idc = ids_v[r, sl]
            mc = mask_v[r, sl]
            pos = plsc.cumsum(mc) - mc + off
            blk = (idc >> 15) << 15
            r15 = idc & jnp.int32(32767)
            row = blk + ((r15 & jnp.int32(8191)) << 2) + (r15 >> 13)
            plsc.store_scatter(ids_v, [rv, pos], row, mask=mc != 0)
            off = off + jnp.sum(mc)
        return off

    CH = 2 * L

    def chunk_copy(r, slot, c):
        co = pl.multiple_of(c * CH, CH)
        return pltpu.make_async_copy(
            emb_hbm.at[ids_v.at[r, pl.ds(co, CH)]],
            buf_v.at[pl.ds(slot * SP, SP)].at[pl.ds(co, CH)],
            sems[slot],
        )

    def issue(r, slot, m):
        nch = (m + CH - 1) // CH

        def ic(c, _):
            chunk_copy(r, slot, c).start()
            return 0

        lax.fori_loop(0, nch, ic, 0)

    def drain(r, slot, m):
        nch = (m + CH - 1) // CH

        def dc(c, _):
            chunk_copy(r, slot, c).wait()
            return 0

        lax.fori_loop(0, nch, dc, 0)

    def addtok(s, carry):
        a0, a1, a2, a3 = carry
        b0 = plsc.bitcast(buf_v[s, pl.ds(0, L)], jnp.bfloat16)
        b1 = plsc.bitcast(buf_v[s, pl.ds(L, L)], jnp.bfloat16)
        l0, h0 = plsc.unpack(b0, format=plsc.PackFormat.INTERLEAVED)
        l1, h1 = plsc.unpack(b1, format=plsc.PackFormat.INTERLEAVED)
        return a0 + l0, a1 + h0, a2 + l1, a3 + h1

    def accum(r, slot, m):
        n8 = m // UN

        def t8(t, carry):
            for u in range(UN):
                carry = addtok(slot * SP + t * UN + u, carry)
            return carry

        z = jnp.zeros((L,), jnp.float32)
        acc = lax.fori_loop(0, n8, t8, (z, z, z, z))

        def t1(s_, carry):
            return addtok(slot * SP + s_, carry)

        a0, a1, a2, a3 = lax.fori_loop(n8 * UN, m, t1, acc)
        out_v[r, pl.ds(0, L)] = a0
        out_v[r, pl.ds(L, L)] = a1
        out_v[r, pl.ds(2 * L, L)] = a2
        out_v[r, pl.ds(3 * L, L)] = a3

    def prep(rnext, slot):
        safe = jnp.where(rnext < RW, rnext, 0)
        m = compact(safe)

        @pl.when(rnext < RW)
        def _():
            issue(rnext, slot, m)

        return m

    NSLOT = 4
    ms = []
    for j in range(NSLOT):
        mj = compact(jnp.int32(j))
        issue(jnp.int32(j), j, mj)
        ms.append(mj)

    def body(i, carry):
        carry = list(carry)
        for j in range(NSLOT):
            r = NSLOT * i + j
            drain(r, j, carry[j])
            accum(r, j, carry[j])
            carry[j] = prep(r + NSLOT, j)
        return tuple(carry)

    lax.fori_loop(0, RW // NSLOT, body, tuple(ms))
    pltpu.sync_copy(out_v, sum_hbm.at[pl.ds(base, RW)])


def _head_body(sum_ref, mask_ref, sm_ref, ss_ref, wt_ref, bias_ref, out_ref):
    cnt = jnp.sum(mask_ref[...].astype(jnp.float32), axis=1, keepdims=True)
    sums = sum_ref[...]
    pooled = sums / jnp.maximum(cnt, 1e-9)
    nrm = jnp.sqrt(jnp.sum(pooled * pooled, axis=1, keepdims=True))
    pooled = pooled / jnp.maximum(nrm, 1e-32)
    scaled = (pooled - sm_ref[...]) / ss_ref[...]
    out_ref[...] = (
        jnp.dot(scaled, wt_ref[...], preferred_element_type=jnp.float32) + bias_ref[...]
    )


_head = pl.pallas_call(
    _head_body,
    out_shape=jax.ShapeDtypeStruct((B, C), jnp.float32),
)

TP = 32768


NB = (VOCAB + TP - 1) // TP
Q = TP // 4


def _repack_body(in_ref, out_ref):
    xb = in_ref[...].astype(jnp.bfloat16)
    packed = pltpu.bitcast(xb, jnp.float32)
    for a in range(4):
        out_ref[:, 32 * a : 32 * (a + 1)] = packed[:, a * Q : (a + 1) * Q].T


_repack = pl.pallas_call(
    _repack_body,
    grid=(NB,),
    in_specs=[pl.BlockSpec((D, TP), lambda i: (0, i))],
    out_specs=pl.BlockSpec((Q, 2 * D), lambda i: (i, 0)),
    out_shape=jax.ShapeDtypeStruct((NB * Q, 2 * D), jnp.float32),
)


def kernel(input_ids, attention_mask, embedding, scaler_mean, scaler_scale, W, b):
    ids = input_ids.astype(jnp.int32)
    mask = attention_mask.astype(jnp.int32)
    ids_p = jnp.pad(ids, ((0, 0), (0, SP - S)))
    mask_p = jnp.pad(mask, ((0, 0), (0, SP - S)))
    emb2 = _repack(embedding.T).reshape(4 * NB * Q, D // 2)
    sums = _sc_pool(ids_p, mask_p, emb2)
    perm = jnp.array(
        [2 * j for j in range(16)] + [2 * j + 1 for j in range(16)]
        + [32 + 2 * j for j in range(16)] + [32 + 2 * j + 1 for j in range(16)],
        dtype=jnp.int32,
    )
    return _head(
        sums,
        mask,
        scaler_mean[perm].reshape(1, D),
        scaler_scale[perm].reshape(1, D),
        W[:, perm].T,
        b.reshape(1, C),
    )


# speedup vs baseline: 1.0076x; 1.0076x over previous
import functools

import jax
import jax.numpy as jnp
from jax import lax
from jax.experimental import pallas as pl
from jax.experimental.pallas import tpu as pltpu
from jax.experimental.pallas import tpu_sc as plsc

B = 4096
VOCAB = 1000000
S = 200
D = 64
C = 2
L = 16
SP = 224
NC = 2
NS = 16
NW = NC * NS
RW = B // NW
NCH = SP // L
UN = 8

_mesh = plsc.VectorSubcoreMesh(
    core_axis_name="c", subcore_axis_name="s", num_cores=NC, num_subcores=NS
)


@functools.partial(
    pl.kernel,
    out_type=jax.ShapeDtypeStruct((B, D), jnp.float32),
    mesh=_mesh,
    scratch_types=[
        pltpu.VMEM((RW, SP), jnp.int32),
        pltpu.VMEM((RW, SP), jnp.int32),
        pltpu.VMEM((4 * SP, D // 2), jnp.float32),
        pltpu.VMEM((RW, D), jnp.float32),
        pltpu.SemaphoreType.DMA,
        pltpu.SemaphoreType.DMA,
        pltpu.SemaphoreType.DMA,
        pltpu.SemaphoreType.DMA,
    ],
    compiler_params=pltpu.CompilerParams(
        use_tc_tiling_on_sc=False, needs_layout_passes=False
    ),
)
def _sc_pool(ids_hbm, mask_hbm, emb_hbm, sum_hbm, ids_v, mask_v, buf_v, out_v,
             sem0, sem1, sem2, sem3):
    wid = lax.axis_index("s") * NC + lax.axis_index("c")
    base = wid * RW
    sems = (sem0, sem1, sem2, sem3)

    pltpu.sync_copy(ids_hbm.at[pl.ds(base, RW)], ids_v)
    pltpu.sync_copy(mask_hbm.at[pl.ds(base, RW)], mask_v)

    def compact(r):
        rv = jnp.broadcast_to(r.astype(jnp.int32), (L,))
        off = jnp.int32(0)
        for k in range(NCH):
            sl = pl.ds(k * L, L)
            idc = ids_v[r, sl]
            mc = mask_v[r, sl]
            pos = plsc.cumsum(mc) - mc + off
            plsc.store_scatter(ids_v, [rv, pos], idc * 4, mask=mc != 0)
            off = off + jnp.sum(mc)
        return off

    CH = 2 * L

    def chunk_copy(r, slot, c):
        co = pl.multiple_of(c * CH, CH)
        return pltpu.make_async_copy(
            emb_hbm.at[ids_v.at[r, pl.ds(co, CH)]],
            buf_v.at[pl.ds(slot * SP, SP)].at[pl.ds(co, CH)],
            sems[slot],
        )

    def issue(r, slot, m):
        nch = (m + CH - 1) // CH

        def ic(c, _):
            chunk_copy(r, slot, c).start()
            return 0

        lax.fori_loop(0, nch, ic, 0)

    def drain(r, slot, m):
        nch = (m + CH - 1) // CH

        def dc(c, _):
            chunk_copy(r, slot, c).wait()
            return 0

        lax.fori_loop(0, nch, dc, 0)

    def addtok(s, carry):
        a0, a1, a2, a3 = carry
        b0 = plsc.bitcast(buf_v[s, pl.ds(0, L)], jnp.bfloat16)
        b1 = plsc.bitcast(buf_v[s, pl.ds(L, L)], jnp.bfloat16)
        l0, h0 = plsc.unpack(b0, format=plsc.PackFormat.INTERLEAVED)
        l1, h1 = plsc.unpack(b1, format=plsc.PackFormat.INTERLEAVED)
        return a0 + l0, a1 + h0, a2 + l1, a3 + h1

    def accum(r, slot, m):
        n8 = m // UN

        def t8(t, carry):
            for u in range(UN):
                carry = addtok(slot * SP + t * UN + u, carry)
            return carry

        z = jnp.zeros((L,), jnp.float32)
        acc = lax.fori_loop(0, n8, t8, (z, z, z, z))

        def t1(s_, carry):
            return addtok(slot * SP + s_, carry)

        a0, a1, a2, a3 = lax.fori_loop(n8 * UN, m, t1, acc)
        out_v[r, pl.ds(0, L)] = a0
        out_v[r, pl.ds(L, L)] = a1
        out_v[r, pl.ds(2 * L, L)] = a2
        out_v[r, pl.ds(3 * L, L)] = a3

    def prep(rnext, slot):
        safe = jnp.where(rnext < RW, rnext, 0)
        m = compact(safe)

        @pl.when(rnext < RW)
        def _():
            issue(rnext, slot, m)

        return m

    NSLOT = 4
    ms = []
    for j in range(NSLOT):
        mj = compact(jnp.int32(j))
        issue(jnp.int32(j), j, mj)
        ms.append(mj)

    def body(i, carry):
        carry = list(carry)
        for j in range(NSLOT):
            r = NSLOT * i + j
            drain(r, j, carry[j])
            accum(r, j, carry[j])
            carry[j] = prep(r + NSLOT, j)
        return tuple(carry)

    lax.fori_loop(0, RW // NSLOT, body, tuple(ms))
    pltpu.sync_copy(out_v, sum_hbm.at[pl.ds(base, RW)])


def _head_body(sum_ref, mask_ref, sm_ref, ss_ref, wt_ref, bias_ref, out_ref):
    cnt = jnp.sum(mask_ref[...].astype(jnp.float32), axis=1, keepdims=True)
    sums = sum_ref[...]
    pooled = sums / jnp.maximum(cnt, 1e-9)
    nrm = jnp.sqrt(jnp.sum(pooled * pooled, axis=1, keepdims=True))
    pooled = pooled / jnp.maximum(nrm, 1e-32)
    scaled = (pooled - sm_ref[...]) / ss_ref[...]
    out_ref[...] = (
        jnp.dot(scaled, wt_ref[...], preferred_element_type=jnp.float32) + bias_ref[...]
    )


_head = pl.pallas_call(
    _head_body,
    out_shape=jax.ShapeDtypeStruct((B, C), jnp.float32),
)

TP = 32768


def _repack_body(in_ref, out_ref):
    xb = in_ref[...].astype(jnp.bfloat16)
    packed = pltpu.bitcast(xb, jnp.float32)
    out_ref[:, : D // 2] = packed.T


_repack = pl.pallas_call(
    _repack_body,
    grid=((VOCAB + TP - 1) // TP,),
    in_specs=[pl.BlockSpec((D, TP), lambda i: (0, i))],
    out_specs=pl.BlockSpec((TP, 2 * D), lambda i: (i, 0)),
    out_shape=jax.ShapeDtypeStruct((VOCAB, 2 * D), jnp.float32),
)


def kernel(input_ids, attention_mask, embedding, scaler_mean, scaler_scale, W, b):
    ids = input_ids.astype(jnp.int32)
    mask = attention_mask.astype(jnp.int32)
    ids_p = jnp.pad(ids, ((0, 0), (0, SP - S)))
    mask_p = jnp.pad(mask, ((0, 0), (0, SP - S)))
    emb2 = _repack(embedding.T).reshape(4 * VOCAB, D // 2)
    sums = _sc_pool(ids_p, mask_p, emb2)
    perm = jnp.array(
        [2 * j for j in range(16)] + [2 * j + 1 for j in range(16)]
        + [32 + 2 * j for j in range(16)] + [32 + 2 * j + 1 for j in range(16)],
        dtype=jnp.int32,
    )
    return _head(
        sums,
        mask,
        scaler_mean[perm].reshape(1, D),
        scaler_scale[perm].reshape(1, D),
        W[:, perm].T,
        b.reshape(1, C),
    )
